# baseline (speedup 1.0000x reference)
import functools

import jax
import jax.numpy as jnp
from jax import lax
from jax.experimental import pallas as pl
from jax.experimental.pallas import tpu as pltpu
from jax.experimental.pallas import tpu_sc as plsc

N = 10000
D = 128
E = 320000

NC = 2
NS = 16
NW = NC * NS

K = 128
CH = 80
EW = CH * K
E_PAD = NW * EW
NB = 2
NQ = 4

N_PAD = 10112
RW = N_PAD // NS
DUMMY = N

_sc_mesh = plsc.VectorSubcoreMesh(
    core_axis_name="c", subcore_axis_name="s", num_cores=NC, num_subcores=NS
)


def _make_sc_agg(with_deg):
    def body(*refs):
        if with_deg:
            (x_hbm, src_hbm, dst_hbm, zrows_hbm, zdeg_hbm, acc_out, deg_out,
             acc_sh, deg_sh, r0, r1, s0, s1, s2, s3, d0, d1, d2, d3,
             ones_v, deg_v, *sems) = refs
        else:
            (x_hbm, src_hbm, dst_hbm, zrows_hbm, acc_out,
             acc_sh, r0, r1, s0, s1, s2, s3, d0, d1, d2, d3, *sems) = refs
        rows = [r0, r1]
        sidx = [s0, s1, s2, s3]
        didx = [d0, d1, d2, d3]
        gsem = sems[0:NB]
        ssem = sems[NB:2 * NB]
        isem = sems[2 * NB:2 * NB + NQ]
        dsem = sems[2 * NB + NQ:] if with_deg else None

        cid = lax.axis_index("c")
        sid = lax.axis_index("s")
        wid = cid * NS + sid

        pltpu.sync_copy(zrows_hbm.at[pl.ds(sid * RW, RW)],
                        acc_sh.at[pl.ds(sid * RW, RW)])
        if with_deg:
            pltpu.sync_copy(zdeg_hbm.at[pl.ds(sid * RW, RW)], deg_v)
            pltpu.sync_copy(deg_v, deg_sh.at[pl.ds(sid * RW, RW)])
            for j in range(K // 16):
                ones_v[pl.ds(j * 16, 16)] = jnp.ones((16,), jnp.float32)

        def i_start(c, q):
            off = (wid * CH + c) * K
            pltpu.async_copy(src_hbm.at[pl.ds(off, K)], sidx[q], isem[q])
            pltpu.async_copy(dst_hbm.at[pl.ds(off, K)], didx[q], isem[q])

        def i_wait(c, q):
            off = (wid * CH + c) * K
            pltpu.make_async_copy(src_hbm.at[pl.ds(off, K)], sidx[q],
                                  isem[q]).wait()
            pltpu.make_async_copy(dst_hbm.at[pl.ds(off, K)], didx[q],
                                  isem[q]).wait()

        def g_start(c, b, q):
            pltpu.async_copy(x_hbm.at[sidx[q]], rows[b], gsem[b])

        def g_wait(c, b, q):
            pltpu.make_async_copy(x_hbm.at[sidx[q]], rows[b], gsem[b]).wait()

        def s_start(c, b, q):
            pltpu.async_copy(rows[b], acc_sh.at[didx[q]], ssem[b], add=True)

        def s_wait(c, b, q):
            pltpu.make_async_copy(rows[b], acc_sh.at[didx[q]], ssem[b]).wait()

        def d_start(c, b, q):
            pltpu.async_copy(ones_v, deg_sh.at[didx[q]], dsem[b], add=True)

        def d_wait(c, b, q):
            pltpu.make_async_copy(ones_v, deg_sh.at[didx[q]], dsem[b]).wait()

        for q in range(NQ - 1):
            i_start(q, q)
        plsc.subcore_barrier()
        i_wait(0, 0)
        g_start(0, 0, 0)

        def step(t, carry):
            for j in range(NQ):
                c = t * NQ + j
                b = j % NB
                q = j
                bp = 1 - b
                qp = (j - 1) % NQ
                qn = (j + 1) % NQ

                g_wait(c, b, q)
                s_start(c, b, q)
                if with_deg:
                    d_start(c, b, q)

                def retire():
                    s_wait(c - 1, bp, qp)
                    if with_deg:
                        d_wait(c - 1, bp, qp)
                if j == 0:
                    pl.when(t > 0)(retire)
                else:
                    retire()

                @pl.when(c + NQ - 1 < CH)
                def _():
                    i_start(c + NQ - 1, qp)

                @pl.when(c + 1 < CH)
                def _():
                    i_wait(c + 1, qn)
                    g_start(c + 1, bp, qn)
            return carry

        lax.fori_loop(0, CH // NQ, step, 0)

        s_wait(CH - 1, (CH - 1) % NB, (CH - 1) % NQ)
        if with_deg:
            d_wait(CH - 1, (CH - 1) % NB, (CH - 1) % NQ)

        plsc.subcore_barrier()

        pltpu.sync_copy(acc_sh.at[pl.ds(sid * RW, RW)],
                        acc_out.at[cid, pl.ds(sid * RW, RW)])
        if with_deg:
            pltpu.sync_copy(deg_sh.at[pl.ds(sid * RW, RW)], deg_v)
            pltpu.sync_copy(deg_v,
                            deg_out.at[pl.ds(cid * N_PAD + sid * RW, RW)])

    out_type = [jax.ShapeDtypeStruct((NC, N_PAD, D), jnp.float32)]
    scratch = [pltpu.VMEM_SHARED((N_PAD, D), jnp.float32)]
    if with_deg:
        out_type.append(jax.ShapeDtypeStruct((NC * N_PAD,), jnp.float32))
        scratch.append(pltpu.VMEM_SHARED((N_PAD,), jnp.float32))
    scratch += [pltpu.VMEM((K, D), jnp.float32) for _ in range(NB)]
    scratch += [pltpu.VMEM((K,), jnp.int32) for _ in range(2 * NQ)]
    if with_deg:
        scratch += [
            pltpu.VMEM((K,), jnp.float32),
            pltpu.VMEM((RW,), jnp.float32),
        ]
    nsem = 2 * NB + NQ + (NB if with_deg else 0)
    scratch += [pltpu.SemaphoreType.DMA for _ in range(nsem)]

    return pl.kernel(
        body,
        out_type=out_type,
        mesh=_sc_mesh,
        scratch_types=scratch,
    )


_sc_agg_deg = _make_sc_agg(True)
_sc_agg = _make_sc_agg(False)


_TC_R = 1000


def _tc_dense_body(acc_ref, deg_ref, h_ref, wn_ref, ws_ref, b_ref, out_ref):
    p = acc_ref[0] + acc_ref[1]
    d = jnp.maximum(deg_ref[0] + deg_ref[1], 1.0)
    agg = p / d
    y = (jnp.dot(agg, wn_ref[...], preferred_element_type=jnp.float32,
                 precision=lax.Precision.HIGHEST)
         + jnp.dot(h_ref[...], ws_ref[...], preferred_element_type=jnp.float32,
                   precision=lax.Precision.HIGHEST)
         + b_ref[...])
    out_ref[...] = jnp.maximum(y, 0.0)


def _tc_dense(acc, deg, h, w_neigh, w_self, b):
    return pl.pallas_call(
        _tc_dense_body,
        grid=(N // _TC_R,),
        in_specs=[
            pl.BlockSpec((NC, _TC_R, D), lambda i: (0, i, 0)),
            pl.BlockSpec((NC, _TC_R, 1), lambda i: (0, i, 0)),
            pl.BlockSpec((_TC_R, D), lambda i: (i, 0)),
            pl.BlockSpec((D, D), lambda i: (0, 0)),
            pl.BlockSpec((D, D), lambda i: (0, 0)),
            pl.BlockSpec((1, D), lambda i: (0, 0)),
        ],
        out_specs=pl.BlockSpec((_TC_R, D), lambda i: (i, 0)),
        out_shape=jax.ShapeDtypeStruct((N, D), jnp.float32),
    )(acc, deg, h, w_neigh, w_self, b)


def kernel(x, edge_index, W_self1, W_neigh1, b1, W_self2, W_neigh2, b2):
    e = edge_index.astype(jnp.int32)
    pad = E_PAD - E
    src = jnp.concatenate([e[0], jnp.zeros((pad,), jnp.int32)])
    dst = jnp.concatenate([e[1], jnp.full((pad,), DUMMY, jnp.int32)])
    zrows = jnp.zeros((N_PAD, D), jnp.float32)
    zdeg = jnp.zeros((N_PAD,), jnp.float32)
    b1r = b1.reshape(1, D)
    b2r = b2.reshape(1, D)

    acc1, deg = _sc_agg_deg(x, src, dst, zrows, zdeg)
    deg3 = deg.reshape(NC, N_PAD, 1)
    h1 = _tc_dense(acc1, deg3, x, W_neigh1, W_self1, b1r)
    (acc2,) = _sc_agg(h1, src, dst, zrows)
    h2 = _tc_dense(acc2, deg3, h1, W_neigh2, W_self2, b2r)
    return h2
